# baseline (speedup 1.0000x reference)
import functools

import jax
import jax.numpy as jnp
from jax import lax
from jax.experimental import pallas as pl
from jax.experimental.pallas import tpu as pltpu
from jax.experimental.pallas import tpu_sc as plsc

_F32 = jnp.float32
NC = 2
NS = 16
NW = NC * NS


def _sc_mesh():
    return plsc.VectorSubcoreMesh(
        core_axis_name="c", subcore_axis_name="s", num_cores=NC,
        num_subcores=NS)


def _fill_1d(ref, n, value):
    v = jnp.full((16,), value, _F32)

    def body(i, carry):
        ref[pl.ds(i * 16, 16)] = v
        return carry

    lax.fori_loop(0, n // 16, body, 0)


def _sc_degree(src, npad):
    e = src.shape[0]
    epw = e // NW
    ch = 2000
    assert epw % ch == 0
    stripe = npad // NS

    @functools.partial(
        pl.kernel,
        out_type=jax.ShapeDtypeStruct((NC, npad), _F32),
        mesh=_sc_mesh(),
        scratch_types=[
            pltpu.VMEM((ch,), _F32),
            pltpu.VMEM((ch,), jnp.int32),
            pltpu.VMEM((stripe,), _F32),
            pltpu.VMEM_SHARED((npad,), _F32),
        ],
    )
    def k(src_hbm, out_hbm, ones_v, idx_v, zero_v, acc_sp):
        c = lax.axis_index("c")
        s = lax.axis_index("s")
        _fill_1d(ones_v, ch, 1.0)
        _fill_1d(zero_v, stripe, 0.0)
        pltpu.sync_copy(zero_v, acc_sp.at[pl.ds(s * stripe, stripe)])
        plsc.subcore_barrier()
        base = (c * NS + s) * epw

        def it(kk, carry):
            pltpu.sync_copy(src_hbm.at[pl.ds(base + kk * ch, ch)], idx_v)
            pltpu.sync_copy(ones_v, acc_sp.at[idx_v], add=True)
            return carry

        lax.fori_loop(0, epw // ch, it, 0)
        plsc.subcore_barrier()
        pltpu.sync_copy(acc_sp.at[pl.ds(s * stripe, stripe)],
                        out_hbm.at[c, pl.ds(s * stripe, stripe)])

    return k(src)


def _sc_accum_rows(zs, src, dst, npad):
    n, h = zs.shape
    e = src.shape[0]
    epw = e // NW
    ch = 176
    nit = epw // ch
    assert epw % ch == 0 and ch % 8 == 0
    stripe = npad // NS
    assert stripe % 128 == 0
    ngrp = -(-nit // 4)

    @functools.partial(
        pl.kernel,
        out_type=jax.ShapeDtypeStruct((NC, npad, h), _F32),
        mesh=_sc_mesh(),
        scratch_types=(
            [pltpu.VMEM((ch, h), _F32)] * 2
            + [pltpu.VMEM((ch,), jnp.int32)] * 4
            + [pltpu.VMEM((ch,), jnp.int32)] * 4
            + [pltpu.VMEM_SHARED((npad, h), _F32)]
            + [pltpu.SemaphoreType.DMA] * 2
            + [pltpu.SemaphoreType.DMA] * 2
            + [pltpu.SemaphoreType.DMA] * 4
        ),
    )
    def k(zs_hbm, src_hbm, dst_hbm, out_hbm, *refs):
        rows = refs[0:2]
        sidx = refs[2:6]
        didx = refs[6:10]
        acc_sp = refs[10]
        gsem = refs[11:13]
        ssem = refs[13:15]
        isem = refs[15:19]
        c = lax.axis_index("c")
        s = lax.axis_index("s")

        zv = jnp.zeros((16,), _F32)

        def zf(i, carry):
            for j in range(h // 16):
                rows[0][i, pl.ds(j * 16, 16)] = zv
            return carry

        lax.fori_loop(0, 128, zf, 0)

        def zc(j, carry):
            pltpu.sync_copy(rows[0].at[pl.ds(0, 128)],
                            acc_sp.at[pl.ds(s * stripe + j * 128, 128)])
            return carry

        lax.fori_loop(0, stripe // 128, zc, 0)
        plsc.subcore_barrier()
        base = (c * NS + s) * epw

        def idx_load(kc, ib):
            e0 = base + kc * ch
            pltpu.async_copy(src_hbm.at[pl.ds(e0, ch)], sidx[ib], isem[ib])
            pltpu.async_copy(dst_hbm.at[pl.ds(e0, ch)], didx[ib], isem[ib])

        def idx_wait(ib):
            pltpu.make_async_copy(
                src_hbm.at[pl.ds(0, ch)], sidx[ib], isem[ib]).wait()
            pltpu.make_async_copy(
                dst_hbm.at[pl.ds(0, ch)], didx[ib], isem[ib]).wait()

        def scat_wait(b):
            pltpu.make_async_copy(
                rows[b], acc_sp.at[didx[b]], ssem[b]).wait()

        for b in range(4):
            idx_load(b, b)
        for b in range(2):
            idx_wait(b)
            pltpu.async_copy(zs_hbm.at[sidx[b]], rows[b], gsem[b])

        def it(kk, carry):
            for b4 in range(4):
                kc = kk * 4 + b4
                b = b4 % 2
                i = b4

                @pl.when(kc < nit)
                def _():
                    pltpu.make_async_copy(
                        zs_hbm.at[sidx[i]], rows[b], gsem[b]).wait()
                    pltpu.async_copy(rows[b], acc_sp.at[didx[i]], ssem[b],
                                     add=True)

                    @pl.when(kc + 2 < nit)
                    def _():
                        i2 = (b4 + 2) % 4
                        idx_wait(i2)
                        scat_wait(b)

                        @pl.when(kc + 4 < nit)
                        def _():
                            idx_load(kc + 4, i)

                        pltpu.async_copy(zs_hbm.at[sidx[i2]], rows[b],
                                         gsem[b])

            return carry

        lax.fori_loop(0, ngrp, it, 0)
        for b in range(2):
            last = nit - 2 + b
            if last >= 0:
                scat_wait(last % 2)
        plsc.subcore_barrier()

        def co(j, carry):
            r0 = s * stripe + j * 128
            pltpu.sync_copy(acc_sp.at[pl.ds(r0, 128)],
                            out_hbm.at[c, pl.ds(r0, 128)])
            return carry

        lax.fori_loop(0, stripe // 128, co, 0)

    return k(zs, src, dst)


def _sc_scalar_final(tbl, src, dst, y0p, dinvp, npad):
    n = tbl.shape[0]
    e = src.shape[0]
    epw = e // NS
    ch = 2000
    nit = epw // ch
    assert epw % ch == 0 and nit % 2 == 0
    stripe = npad // NS
    stripe2 = npad // NW
    assert stripe2 % 16 == 0

    @functools.partial(
        pl.kernel,
        out_type=jax.ShapeDtypeStruct((npad,), _F32),
        mesh=_sc_mesh(),
        scratch_types=[
            pltpu.VMEM((ch,), _F32),
            pltpu.VMEM((ch,), _F32),
            pltpu.VMEM((ch,), jnp.int32),
            pltpu.VMEM((ch,), jnp.int32),
            pltpu.VMEM((ch,), jnp.int32),
            pltpu.VMEM((ch,), jnp.int32),
            pltpu.VMEM((stripe,), _F32),
            pltpu.VMEM((stripe2,), _F32),
            pltpu.VMEM((stripe2,), _F32),
            pltpu.VMEM((stripe2,), _F32),
            pltpu.VMEM_SHARED((npad,), _F32),
            pltpu.VMEM_SHARED((n,), _F32),
            pltpu.SemaphoreType.DMA,
            pltpu.SemaphoreType.DMA,
        ],
    )
    def k(tbl_hbm, src_hbm, dst_hbm, y0_hbm, dinv_hbm, out_hbm, vals0_v,
          vals1_v, si0_v, si1_v, di0_v, di1_v, zero_v, y0_v, dinv_v,
          out_v, acc_sp, tbl_sp, sem0, sem1):
        c = lax.axis_index("c")
        s = lax.axis_index("s")
        vals = (vals0_v, vals1_v)
        sidx = (si0_v, si1_v)
        didx = (di0_v, di1_v)
        sems = (sem0, sem1)
        _fill_1d(zero_v, stripe, 0.0)
        pltpu.sync_copy(zero_v, acc_sp.at[pl.ds(s * stripe, stripe)])

        @pl.when(s == 0)
        def _():
            pltpu.sync_copy(tbl_hbm, tbl_sp)

        plsc.subcore_barrier()
        base = s * epw

        for b in range(2):
            pltpu.sync_copy(src_hbm.at[pl.ds(base + b * ch, ch)], sidx[b])
            pltpu.sync_copy(dst_hbm.at[pl.ds(base + b * ch, ch)], didx[b])
            pltpu.async_copy(tbl_sp.at[sidx[b]], vals[b], sems[b])

        def it(kk, carry):
            for b in range(2):
                kc = kk * 2 + b
                pltpu.make_async_copy(
                    tbl_sp.at[sidx[b]], vals[b], sems[b]).wait()
                pltpu.sync_copy(vals[b], acc_sp.at[didx[b]], add=True)

                @pl.when(kc + 2 < nit)
                def _():
                    e0 = base + (kc + 2) * ch
                    pltpu.sync_copy(src_hbm.at[pl.ds(e0, ch)], sidx[b])
                    pltpu.sync_copy(dst_hbm.at[pl.ds(e0, ch)], didx[b])
                    pltpu.async_copy(tbl_sp.at[sidx[b]], vals[b],
                                     sems[b])

            return carry

        lax.fori_loop(0, nit // 2, it, 0)
        plsc.subcore_barrier()
        g0 = (c * NS + s) * stripe2
        pltpu.sync_copy(acc_sp.at[pl.ds(g0, stripe2)],
                        zero_v.at[pl.ds(0, stripe2)])
        pltpu.sync_copy(y0_hbm.at[pl.ds(g0, stripe2)], y0_v)
        pltpu.sync_copy(dinv_hbm.at[pl.ds(g0, stripe2)], dinv_v)

        def sg(i, carry):
            sl = pl.ds(i * 16, 16)
            t = y0_v[sl] - dinv_v[sl] * zero_v[sl]
            out_v[sl] = 1.0 / (1.0 + jnp.exp(-t))
            return carry

        lax.fori_loop(0, stripe2 // 16, sg, 0)
        pltpu.sync_copy(out_v, out_hbm.at[pl.ds(g0, stripe2)])

    return k(tbl, src, dst, y0p, dinvp)


def _tc_pre(x, deg2, w0, w1, b1):
    n, d = x.shape
    h = w0.shape[1]

    def body(x_ref, deg2_ref, w0_ref, w1_ref, b1_ref, dinv_ref, a_ref,
             zs_ref):
        deg = deg2_ref[0, 0:n] + deg2_ref[1, 0:n]
        dinv = jnp.where(deg > 0, lax.rsqrt(jnp.maximum(deg, 1e-12)), 0.0)
        dinv_ref[...] = dinv
        xv = x_ref[...]
        a_ref[...] = (jnp.dot(xv, w0_ref[...], preferred_element_type=_F32)
                      + b1_ref[...][None, :])
        zs_ref[...] = dinv[:, None] * jnp.dot(
            xv, w1_ref[...], preferred_element_type=_F32)

    return pl.pallas_call(
        body,
        out_shape=(
            jax.ShapeDtypeStruct((n,), _F32),
            jax.ShapeDtypeStruct((n, h), _F32),
            jax.ShapeDtypeStruct((n, h), _F32),
        ),
    )(x, deg2, w0, w1, b1)


def _tc_mid(a, acc2, dinv, w0b, w1b, b2):
    n, h = a.shape

    def body(a_ref, acc2_ref, dinv_ref, w0b_ref, w1b_ref, b2_ref, y0_ref,
             y1s_ref):
        acc = acc2_ref[0, 0:n, :] + acc2_ref[1, 0:n, :]
        dinv = dinv_ref[...]
        hv = jnp.maximum(a_ref[...] - dinv[:, None] * acc, 0.0)
        y0 = jnp.dot(hv, w0b_ref[...], preferred_element_type=_F32)[:, 0]
        y1 = jnp.dot(hv, w1b_ref[...], preferred_element_type=_F32)[:, 0]
        y0_ref[...] = y0 + b2_ref[0]
        y1s_ref[...] = dinv * y1

    return pl.pallas_call(
        body,
        out_shape=(
            jax.ShapeDtypeStruct((n,), _F32),
            jax.ShapeDtypeStruct((n,), _F32),
        ),
    )(a, acc2, dinv, w0b, w1b, b2)


def kernel(x, edge_index, W0a, W1a, b1, W0b, W1b, b2):
    n, d = x.shape
    src = edge_index[0]
    dst = edge_index[1]
    npad = -(-n // 2048) * 2048

    epw_p = -(-(edge_index.shape[1] // NW) // 176) * 176
    ep = epw_p * NW
    npe = ep - edge_index.shape[1]
    pad_ar = jnp.arange(npe, dtype=jnp.int32)
    src_p = jnp.concatenate([src, pad_ar % n])
    dst_p = jnp.concatenate([dst, n + pad_ar % (npad - n)])

    deg2 = _sc_degree(src, npad)
    dinv, a, zs = _tc_pre(x, deg2, W0a, W1a, b1)
    acc2 = _sc_accum_rows(zs, src_p, dst_p, npad)
    y0, y1s = _tc_mid(a, acc2, dinv, W0b, W1b, b2)
    y0p = jnp.pad(y0, (0, npad - n))
    dinvp = jnp.pad(dinv, (0, npad - n))
    out = _sc_scalar_final(y1s, src, dst, y0p, dinvp, npad)
    return out[0:n].reshape(n, 1)


# speedup vs baseline: 1.1565x; 1.1565x over previous
import functools

import jax
import jax.numpy as jnp
from jax import lax
from jax.experimental import pallas as pl
from jax.experimental.pallas import tpu as pltpu
from jax.experimental.pallas import tpu_sc as plsc

_F32 = jnp.float32
NC = 2
NS = 16
NW = NC * NS


def _sc_mesh():
    return plsc.VectorSubcoreMesh(
        core_axis_name="c", subcore_axis_name="s", num_cores=NC,
        num_subcores=NS)


def _fill_1d(ref, n, value):
    v = jnp.full((16,), value, _F32)

    def body(i, carry):
        ref[pl.ds(i * 16, 16)] = v
        return carry

    lax.fori_loop(0, n // 16, body, 0)


def _sc_degree(src, npad):
    e = src.shape[0]
    epw = e // NW
    ch = 2000
    assert epw % ch == 0
    stripe = npad // NS

    @functools.partial(
        pl.kernel,
        out_type=jax.ShapeDtypeStruct((NC, npad), _F32),
        mesh=_sc_mesh(),
        scratch_types=[
            pltpu.VMEM((ch,), _F32),
            pltpu.VMEM((ch,), jnp.int32),
            pltpu.VMEM((ch,), jnp.int32),
            pltpu.VMEM((stripe,), _F32),
            pltpu.VMEM_SHARED((npad,), _F32),
            pltpu.SemaphoreType.DMA,
            pltpu.SemaphoreType.DMA,
        ],
    )
    def k(src_hbm, out_hbm, ones_v, idx0_v, idx1_v, zero_v, acc_sp,
          sem0, sem1):
        c = lax.axis_index("c")
        s = lax.axis_index("s")
        idx = (idx0_v, idx1_v)
        sems = (sem0, sem1)
        nit = epw // ch
        _fill_1d(ones_v, ch, 1.0)
        _fill_1d(zero_v, stripe, 0.0)
        base = (c * NS + s) * epw
        for b in range(2):
            pltpu.async_copy(src_hbm.at[pl.ds(base + b * ch, ch)], idx[b],
                             sems[b])
        pltpu.sync_copy(zero_v, acc_sp.at[pl.ds(s * stripe, stripe)])
        plsc.subcore_barrier()

        def it(kk, carry):
            for b in range(2):
                kc = kk * 2 + b

                @pl.when(kc < nit)
                def _():
                    pltpu.make_async_copy(
                        src_hbm.at[pl.ds(0, ch)], idx[b], sems[b]).wait()
                    pltpu.sync_copy(ones_v, acc_sp.at[idx[b]], add=True)

                    @pl.when(kc + 2 < nit)
                    def _():
                        pltpu.async_copy(
                            src_hbm.at[pl.ds(base + (kc + 2) * ch, ch)],
                            idx[b], sems[b])

            return carry

        lax.fori_loop(0, -(-nit // 2), it, 0)
        plsc.subcore_barrier()
        pltpu.sync_copy(acc_sp.at[pl.ds(s * stripe, stripe)],
                        out_hbm.at[c, pl.ds(s * stripe, stripe)])

    return k(src)


def _sc_accum_rows(zs, src, dst, npad):
    n, h = zs.shape
    e = src.shape[0]
    epw = e // NW
    ch = 80
    nit = epw // ch
    assert epw % ch == 0 and ch % 8 == 0
    stripe = npad // NS
    assert stripe % ch == 0 and stripe % 128 == 0
    ngrp = -(-nit // 8)

    @functools.partial(
        pl.kernel,
        out_type=jax.ShapeDtypeStruct((NC, npad, h), _F32),
        mesh=_sc_mesh(),
        scratch_types=(
            [pltpu.VMEM((ch, h), _F32)] * 4
            + [pltpu.VMEM((ch,), jnp.int32)] * 8
            + [pltpu.VMEM((ch,), jnp.int32)] * 8
            + [pltpu.VMEM_SHARED((npad, h), _F32)]
            + [pltpu.SemaphoreType.DMA] * 4
            + [pltpu.SemaphoreType.DMA] * 4
            + [pltpu.SemaphoreType.DMA] * 8
        ),
    )
    def k(zs_hbm, src_hbm, dst_hbm, out_hbm, *refs):
        rows = refs[0:4]
        sidx = refs[4:12]
        didx = refs[12:20]
        acc_sp = refs[20]
        gsem = refs[21:25]
        ssem = refs[25:29]
        isem = refs[29:37]
        c = lax.axis_index("c")
        s = lax.axis_index("s")

        zv = jnp.zeros((16,), _F32)

        def zf(i, carry):
            for j in range(h // 16):
                rows[0][i, pl.ds(j * 16, 16)] = zv
            return carry

        lax.fori_loop(0, ch, zf, 0)

        def zc(j, carry):
            pltpu.sync_copy(rows[0],
                            acc_sp.at[pl.ds(s * stripe + j * ch, ch)])
            return carry

        lax.fori_loop(0, stripe // ch, zc, 0)
        plsc.subcore_barrier()
        base = (c * NS + s) * epw

        def idx_load(kc, ib):
            e0 = base + kc * ch
            pltpu.async_copy(src_hbm.at[pl.ds(e0, ch)], sidx[ib], isem[ib])
            pltpu.async_copy(dst_hbm.at[pl.ds(e0, ch)], didx[ib], isem[ib])

        def idx_wait(ib):
            pltpu.make_async_copy(
                src_hbm.at[pl.ds(0, ch)], sidx[ib], isem[ib]).wait()
            pltpu.make_async_copy(
                dst_hbm.at[pl.ds(0, ch)], didx[ib], isem[ib]).wait()

        def scat_wait(b):
            pltpu.make_async_copy(
                rows[b], acc_sp.at[didx[b]], ssem[b]).wait()

        for b in range(8):
            idx_load(b, b)
        for b in range(4):
            idx_wait(b)
            pltpu.async_copy(zs_hbm.at[sidx[b]], rows[b], gsem[b])

        def it(kk, carry):
            for b8 in range(8):
                kc = kk * 8 + b8
                b = b8 % 4
                i = b8

                @pl.when(kc < nit)
                def _():
                    pltpu.make_async_copy(
                        zs_hbm.at[sidx[i]], rows[b], gsem[b]).wait()
                    pltpu.async_copy(rows[b], acc_sp.at[didx[i]], ssem[b],
                                     add=True)

                    @pl.when(kc + 4 < nit)
                    def _():
                        i2 = (b8 + 4) % 8
                        idx_wait(i2)
                        scat_wait(b)

                        @pl.when(kc + 8 < nit)
                        def _():
                            idx_load(kc + 8, i)

                        pltpu.async_copy(zs_hbm.at[sidx[i2]], rows[b],
                                         gsem[b])

            return carry

        lax.fori_loop(0, ngrp, it, 0)
        for b in range(4):
            last = nit - 4 + b
            if last >= 0:
                scat_wait(last % 4)
        plsc.subcore_barrier()

        def co(j, carry):
            r0 = s * stripe + j * 128
            pltpu.sync_copy(acc_sp.at[pl.ds(r0, 128)],
                            out_hbm.at[c, pl.ds(r0, 128)])
            return carry

        lax.fori_loop(0, stripe // 128, co, 0)

    return k(zs, src, dst)


def _sc_scalar_final(tbl, src, dst, y0p, dinvp, npad):
    n = tbl.shape[0]
    e = src.shape[0]
    epw = e // NS
    ch = 2000
    nit = epw // ch
    assert epw % ch == 0
    stripe = npad // NS
    stripe2 = npad // NW
    assert stripe2 % 16 == 0

    @functools.partial(
        pl.kernel,
        out_type=jax.ShapeDtypeStruct((npad,), _F32),
        mesh=_sc_mesh(),
        scratch_types=[
            pltpu.VMEM((ch,), _F32),
            pltpu.VMEM((ch,), _F32),
            pltpu.VMEM((ch,), jnp.int32),
            pltpu.VMEM((ch,), jnp.int32),
            pltpu.VMEM((ch,), jnp.int32),
            pltpu.VMEM((ch,), jnp.int32),
            pltpu.VMEM((ch,), jnp.int32),
            pltpu.VMEM((ch,), jnp.int32),
            pltpu.VMEM((ch,), jnp.int32),
            pltpu.VMEM((ch,), jnp.int32),
            pltpu.VMEM((stripe,), _F32),
            pltpu.VMEM((stripe2,), _F32),
            pltpu.VMEM((stripe2,), _F32),
            pltpu.VMEM((stripe2,), _F32),
            pltpu.VMEM_SHARED((npad,), _F32),
            pltpu.VMEM_SHARED((n,), _F32),
            pltpu.SemaphoreType.DMA,
            pltpu.SemaphoreType.DMA,
            pltpu.SemaphoreType.DMA,
            pltpu.SemaphoreType.DMA,
            pltpu.SemaphoreType.DMA,
            pltpu.SemaphoreType.DMA,
        ],
    )
    def k(tbl_hbm, src_hbm, dst_hbm, y0_hbm, dinv_hbm, out_hbm, vals0_v,
          vals1_v, si0_v, si1_v, si2_v, si3_v, di0_v, di1_v, di2_v, di3_v,
          zero_v, y0_v, dinv_v, out_v, acc_sp, tbl_sp, sem0, sem1,
          is0, is1, is2, is3):
        c = lax.axis_index("c")
        s = lax.axis_index("s")
        vals = (vals0_v, vals1_v)
        sidx = (si0_v, si1_v, si2_v, si3_v)
        didx = (di0_v, di1_v, di2_v, di3_v)
        sems = (sem0, sem1)
        isems = (is0, is1, is2, is3)
        _fill_1d(zero_v, stripe, 0.0)
        pltpu.sync_copy(zero_v, acc_sp.at[pl.ds(s * stripe, stripe)])

        @pl.when(s == 0)
        def _():
            pltpu.sync_copy(tbl_hbm, tbl_sp)

        plsc.subcore_barrier()
        base = s * epw

        def idx_load(kc, b):
            e0 = base + kc * ch
            pltpu.async_copy(src_hbm.at[pl.ds(e0, ch)], sidx[b], isems[b])
            pltpu.async_copy(dst_hbm.at[pl.ds(e0, ch)], didx[b], isems[b])

        def idx_wait(b):
            pltpu.make_async_copy(
                src_hbm.at[pl.ds(0, ch)], sidx[b], isems[b]).wait()
            pltpu.make_async_copy(
                dst_hbm.at[pl.ds(0, ch)], didx[b], isems[b]).wait()

        for b in range(4):
            idx_load(b, b)
        for b in range(2):
            idx_wait(b)
            pltpu.async_copy(tbl_sp.at[sidx[b]], vals[b], sems[b])

        def it(kk, carry):
            for b4 in range(4):
                kc = kk * 4 + b4
                b = b4 % 2

                @pl.when(kc < nit)
                def _():
                    pltpu.make_async_copy(
                        tbl_sp.at[sidx[b4]], vals[b], sems[b]).wait()
                    pltpu.sync_copy(vals[b], acc_sp.at[didx[b4]], add=True)

                    @pl.when(kc + 2 < nit)
                    def _():
                        b2 = (b4 + 2) % 4
                        idx_wait(b2)

                        @pl.when(kc + 4 < nit)
                        def _():
                            idx_load(kc + 4, b4)

                        pltpu.async_copy(tbl_sp.at[sidx[b2]], vals[b],
                                         sems[b])

            return carry

        lax.fori_loop(0, -(-nit // 4), it, 0)
        plsc.subcore_barrier()
        g0 = (c * NS + s) * stripe2
        pltpu.sync_copy(acc_sp.at[pl.ds(g0, stripe2)],
                        zero_v.at[pl.ds(0, stripe2)])
        pltpu.sync_copy(y0_hbm.at[pl.ds(g0, stripe2)], y0_v)
        pltpu.sync_copy(dinv_hbm.at[pl.ds(g0, stripe2)], dinv_v)

        def sg(i, carry):
            sl = pl.ds(i * 16, 16)
            t = y0_v[sl] - dinv_v[sl] * zero_v[sl]
            out_v[sl] = 1.0 / (1.0 + jnp.exp(-t))
            return carry

        lax.fori_loop(0, stripe2 // 16, sg, 0)
        pltpu.sync_copy(out_v, out_hbm.at[pl.ds(g0, stripe2)])

    return k(tbl, src, dst, y0p, dinvp)


def _tc_pre(x, deg2, w1):
    n, d = x.shape
    h = w1.shape[1]

    def body(x_ref, deg2_ref, w1_ref, dinv_ref, zs_ref):
        deg = deg2_ref[0, 0:n] + deg2_ref[1, 0:n]
        dinv = jnp.where(deg > 0, lax.rsqrt(jnp.maximum(deg, 1e-12)), 0.0)
        dinv_ref[...] = dinv
        zs_ref[...] = dinv[:, None] * jnp.dot(
            x_ref[...], w1_ref[...], preferred_element_type=_F32)

    return pl.pallas_call(
        body,
        out_shape=(
            jax.ShapeDtypeStruct((n,), _F32),
            jax.ShapeDtypeStruct((n, h), _F32),
        ),
    )(x, deg2, w1)


def _tc_mid(x, w0a, b1, acc2, dinv, w0b, w1b, b2):
    n, h = x.shape[0], w0a.shape[1]

    def body(x_ref, w0a_ref, b1_ref, acc2_ref, dinv_ref, w0b_ref, w1b_ref,
             b2_ref, y0_ref, y1s_ref):
        acc = acc2_ref[0, 0:n, :] + acc2_ref[1, 0:n, :]
        dinv = dinv_ref[...]
        av = (jnp.dot(x_ref[...], w0a_ref[...], preferred_element_type=_F32)
              + b1_ref[...][None, :])
        hv = jnp.maximum(av - dinv[:, None] * acc, 0.0)
        y0 = jnp.dot(hv, w0b_ref[...], preferred_element_type=_F32)[:, 0]
        y1 = jnp.dot(hv, w1b_ref[...], preferred_element_type=_F32)[:, 0]
        y0_ref[...] = y0 + b2_ref[0]
        y1s_ref[...] = dinv * y1

    return pl.pallas_call(
        body,
        out_shape=(
            jax.ShapeDtypeStruct((n,), _F32),
            jax.ShapeDtypeStruct((n,), _F32),
        ),
    )(x, w0a, b1, acc2, dinv, w0b, w1b, b2)


def kernel(x, edge_index, W0a, W1a, b1, W0b, W1b, b2):
    n, d = x.shape
    src = edge_index[0]
    dst = edge_index[1]
    npad = -(-n // 2048) * 2048

    deg2 = _sc_degree(src, npad)
    dinv, zs = _tc_pre(x, deg2, W1a)
    acc2 = _sc_accum_rows(zs, src, dst, npad)
    y0, y1s = _tc_mid(x, W0a, b1, acc2, dinv, W0b, W1b, b2)
    y0p = jnp.pad(y0, (0, npad - n))
    dinvp = jnp.pad(dinv, (0, npad - n))
    out = _sc_scalar_final(y1s, src, dst, y0p, dinvp, npad)
    return out[0:n].reshape(n, 1)
